# baseline (speedup 1.0000x reference)
import functools
import math

import jax
import jax.numpy as jnp
from jax import lax
from jax.experimental import pallas as pl
from jax.experimental.pallas import tpu as pltpu
from jax.experimental.pallas import tpu_sc as plsc

D = 128
SCALE = math.sqrt(float(D))
LANES = 16
NC, NS = 2, 16
NW = NC * NS

B_TOTAL = 1024 * 200
PER_W = B_TOTAL // NW
CHUNK = 128
NCH = PER_W // CHUNK


def _emb_body(seqs_hbm, table_hbm, out_hbm, idx_v, rows_v, sem):
    wid = lax.axis_index("s") * NC + lax.axis_index("c")
    base = wid * PER_W

    def chunk_body(k, carry):
        row0 = base + k * CHUNK
        pltpu.sync_copy(seqs_hbm.at[pl.ds(row0, CHUNK)], idx_v)
        pltpu.async_copy(table_hbm.at[idx_v], rows_v, sem).wait()

        def scale_row(r, c):
            for s in range(D // LANES):
                sl = pl.ds(s * LANES, LANES)
                rows_v[r, sl] = rows_v[r, sl] * SCALE
            return c

        lax.fori_loop(0, CHUNK, scale_row, 0)
        pltpu.sync_copy(rows_v, out_hbm.at[pl.ds(row0, CHUNK)])
        return carry

    lax.fori_loop(0, NCH, chunk_body, 0)


_emb_lookup = functools.partial(
    pl.kernel,
    out_type=jax.ShapeDtypeStruct((B_TOTAL, D), jnp.float32),
    mesh=plsc.VectorSubcoreMesh(core_axis_name="c", subcore_axis_name="s"),
    scratch_types=[
        pltpu.VMEM((CHUNK,), jnp.int32),
        pltpu.VMEM((CHUNK, D), jnp.float32),
        pltpu.SemaphoreType.DMA,
    ],
)(_emb_body)


def _mask_body(lens_ref, out_ref):
    pos = lax.broadcasted_iota(jnp.int32, out_ref.shape, 1)
    valid = pos < lens_ref[:]
    out_ref[:] = jnp.where(valid, jnp.float32(0.0), jnp.float32(-jnp.inf))


def kernel(seqs, seq_lens, embed_table):
    bsz, seq_len = seqs.shape
    flat = seqs.reshape(-1).astype(jnp.int32)
    emb = _emb_lookup(flat, embed_table)
    mask = pl.pallas_call(
        _mask_body,
        out_shape=jax.ShapeDtypeStruct((bsz, seq_len), jnp.float32),
    )(seq_lens.reshape(bsz, 1))
    return emb.reshape(bsz, seq_len, D), mask
